# baseline (speedup 1.0000x reference)
import functools

import jax
import jax.numpy as jnp
from jax import lax
from jax.experimental import pallas as pl
from jax.experimental.pallas import tpu as pltpu
from jax.experimental.pallas import tpu_sc as plsc

_CROP_RATIO = 0.6
_MIN_LENGTH = 3
_MASK_RATIO = 0.3
_REORDER_RATIO = 0.5
_MIN_W = 2
_MAX_W = 5
_COLS = 2048
_CHUNK = 128


def _splat(v, i):
    idx = jnp.full((16, 1), i, jnp.int32)
    dn = lax.GatherDimensionNumbers(
        offset_dims=(), collapsed_slice_dims=(0,), start_index_map=(0,))
    return lax.gather(v, idx, dn, slice_sizes=(1,),
                      mode=lax.GatherScatterMode.PROMISE_IN_BOUNDS)


def _crop_sc_body(seq_hbm, start_hbm, clen_hbm, out_hbm,
                  inbuf, outbuf, startv, clenv):
    B, L = seq_hbm.shape
    nw = 32
    rows_pw = B // nw
    wid = lax.axis_index("s") * 2 + lax.axis_index("c")
    wbase = wid * rows_pw
    pltpu.sync_copy(start_hbm.at[pl.ds(wbase, rows_pw)], startv)
    pltpu.sync_copy(clen_hbm.at[pl.ds(wbase, rows_pw)], clenv)
    lane0 = lax.iota(jnp.int32, 16)
    nsteps = (L + 15) // 16

    def do_chunk(ci, _):
        cbase = wbase + ci * _CHUNK
        pltpu.sync_copy(seq_hbm.at[pl.ds(cbase, _CHUNK)], inbuf)

        def do_group(gi, __):
            st16 = startv[pl.ds(ci * _CHUNK + gi * 16, 16)]
            cl16 = clenv[pl.ds(ci * _CHUNK + gi * 16, 16)]
            for rr in range(16):
                sv = _splat(st16, rr)
                cv = _splat(cl16, rr)
                rv = jnp.full((16,), gi * 16 + rr, jnp.int32)
                for i in range(nsteps):
                    lane = lane0 + 16 * i
                    col = jnp.minimum(sv + lane, L - 1)
                    g = plsc.load_gather(inbuf, [rv, col])
                    outv = jnp.where(lane < cv, g, 0)
                    if 16 * (i + 1) <= L:
                        plsc.store_scatter(outbuf, [rv, lane], outv)
                    else:
                        plsc.store_scatter(outbuf, [rv, lane], outv,
                                           mask=lane < L)
            return __

        lax.fori_loop(0, _CHUNK // 16, do_group, 0)
        pltpu.sync_copy(outbuf, out_hbm.at[pl.ds(cbase, _CHUNK)])
        return _

    lax.fori_loop(0, rows_pw // _CHUNK, do_chunk, 0)


def _crop_sc(item_seq, start_eff, clen_eff):
    B, L = item_seq.shape
    mesh = plsc.VectorSubcoreMesh(core_axis_name="c", subcore_axis_name="s")
    f = functools.partial(
        pl.kernel,
        mesh=mesh,
        compiler_params=pltpu.CompilerParams(needs_layout_passes=False),
        out_type=jax.ShapeDtypeStruct((B, L), jnp.int32),
        scratch_types=[
            pltpu.VMEM((_CHUNK, L), jnp.int32),
            pltpu.VMEM((_CHUNK, L), jnp.int32),
            pltpu.VMEM((B // 32,), jnp.int32),
            pltpu.VMEM((B // 32,), jnp.int32),
        ],
    )(_crop_sc_body)
    return f(item_seq, start_eff, clen_eff)


def _aug_body(seq_ref, len_ref, sc_ref, uw_ref, us_ref, r8_ref,
              ua_ref, mask_ref, reord_ref):
    seq = seq_ref[...]
    lens = len_ref[...]
    L, C = seq.shape
    pos = lax.broadcasted_iota(jnp.int32, (L, C), 0)
    zero = jnp.zeros_like(seq)

    valid = (seq != 0) & (pos < lens)
    vi = valid.astype(jnp.int32)
    n_valid = jnp.sum(vi, axis=0, keepdims=True)
    num_to_mask = jnp.minimum(
        jnp.maximum(1, (n_valid.astype(jnp.float32) * _MASK_RATIO).astype(jnp.int32)),
        n_valid)
    sc = jnp.where(valid, sc_ref[...], 2.0)
    kp1 = jnp.clip(num_to_mask - 1, 0, L - 1) + 1
    lo = jnp.zeros_like(lens)
    hi = jnp.full_like(lens, 1 << 23)
    for _ in range(24):
        mid = (lo + hi) >> 1
        t = lax.bitcast_convert_type(mid + 0x3F800000, jnp.float32) - 1.0
        cnt = jnp.sum((sc <= t).astype(jnp.int32), axis=0, keepdims=True)
        ge = cnt >= kp1
        hi = jnp.where(ge, mid, hi)
        lo = jnp.where(ge, lo, mid + 1)
    thresh = lax.bitcast_convert_type(hi + 0x3F800000, jnp.float32) - 1.0
    apply_m = (lens > 1) & (n_valid > 0)
    to_mask = valid & (sc <= thresh)
    mask_ref[...] = jnp.where(apply_m & to_mask, zero, seq)

    max_possible = jnp.minimum(n_valid, _MAX_W)
    w = _MIN_W + jnp.floor(
        uw_ref[...] * jnp.maximum(max_possible - _MIN_W + 1, 1).astype(jnp.float32)
    ).astype(jnp.int32)
    w = jnp.clip(w, _MIN_W, jnp.maximum(max_possible, _MIN_W))
    max_start2 = jnp.maximum(n_valid - w + 1, 1)
    s = jnp.minimum(
        jnp.floor(us_ref[...] * max_start2.astype(jnp.float32)).astype(jnp.int32),
        max_start2 - 1)
    applied = (ua_ref[...] <= _REORDER_RATIO) & (lens > _MIN_W) & (n_valid >= _MIN_W)

    c = vi
    sh = 1
    while sh < L:
        moved = jnp.concatenate(
            [jnp.zeros((sh, C), jnp.int32), c[:L - sh, :]], axis=0)
        c = c + moved
        sh <<= 1
    excl = c - vi

    posk, valk = [], []
    for k in range(_MAX_W):
        hit = valid & (excl == s + k)
        posk.append(jnp.sum(jnp.where(hit, pos, 0), axis=0, keepdims=True))
        valk.append(jnp.sum(jnp.where(hit, seq, 0), axis=0, keepdims=True))

    r8 = r8_ref[...]
    rk = [jnp.where(k < w, r8[k:k + 1, :], 2.0) for k in range(_MAX_W)]
    ranks = []
    for i in range(_MAX_W):
        acc = jnp.zeros_like(lens)
        for j in range(_MAX_W):
            if j == i:
                continue
            cmp = (rk[j] <= rk[i]) if j < i else (rk[j] < rk[i])
            acc = acc + cmp.astype(jnp.int32)
        ranks.append(acc)

    out_r = seq
    for p in range(_MAX_W):
        vsrc = jnp.zeros_like(lens)
        for i in range(_MAX_W):
            vsrc = vsrc + jnp.where(ranks[i] == p, valk[i], 0)
        cond = applied & (p < w) & (pos == posk[p])
        out_r = jnp.where(cond, vsrc, out_r)
    reord_ref[...] = out_r


def kernel(item_seq, item_seq_len):
    B, L = item_seq.shape
    C = _COLS

    key = jax.random.key(42)
    kc, km, kr = jax.random.split(key, 3)
    u_crop = jax.random.uniform(kc, (B,))
    scores = jax.random.uniform(km, (B, L))
    k1, k2, k3, k4 = jax.random.split(kr, 4)
    u_w = jax.random.uniform(k1, (B,))
    u_s = jax.random.uniform(k2, (B,))
    r = jax.random.uniform(k3, (B, _MAX_W))
    u_apply = jax.random.uniform(k4, (B,))
    r8 = jnp.pad(r.T, ((0, 8 - _MAX_W), (0, 0)), constant_values=2.0)

    lens1 = item_seq_len.astype(jnp.int32)
    crop_len = jnp.maximum(_MIN_LENGTH, (lens1.astype(jnp.float32) * _CROP_RATIO)
                           .astype(jnp.int32))
    crop_len = jnp.minimum(crop_len, lens1)
    max_start = jnp.maximum(lens1 - crop_len + 1, 1)
    start = jnp.minimum(
        jnp.floor(u_crop * max_start.astype(jnp.float32)).astype(jnp.int32),
        max_start - 1)
    apply_c = lens1 > _MIN_LENGTH
    start_eff = jnp.where(apply_c, start, 0)
    clen_eff = jnp.where(apply_c, crop_len, L)
    cl = jnp.where(apply_c, crop_len, lens1)

    cs = _crop_sc(item_seq, start_eff, clen_eff)

    seq_t = item_seq.T
    scores_t = scores.T
    lens = lens1.reshape(1, B)
    row = lambda x: x.reshape(1, B)

    big_spec = pl.BlockSpec((L, C), lambda i: (0, i))
    one_spec = pl.BlockSpec((1, C), lambda i: (0, i))
    r8_spec = pl.BlockSpec((8, C), lambda i: (0, i))

    ms, rs = pl.pallas_call(
        _aug_body,
        grid=(B // C,),
        in_specs=[big_spec, one_spec, big_spec, one_spec,
                  one_spec, r8_spec, one_spec],
        out_specs=[big_spec, big_spec],
        out_shape=[
            jax.ShapeDtypeStruct((L, B), jnp.int32),
            jax.ShapeDtypeStruct((L, B), jnp.int32),
        ],
        compiler_params=pltpu.CompilerParams(
            dimension_semantics=("parallel",)),
    )(seq_t, lens, scores_t, row(u_w), row(u_s), r8, row(u_apply))

    cs, _ = lax.optimization_barrier((cs, ms[0, 0]))

    return cs, cl, ms.T, lens1, rs.T, lens1


# speedup vs baseline: 1.0243x; 1.0243x over previous
import functools

import jax
import jax.numpy as jnp
from jax import lax
from jax.experimental import pallas as pl
from jax.experimental.pallas import tpu as pltpu
from jax.experimental.pallas import tpu_sc as plsc

_CROP_RATIO = 0.6
_MIN_LENGTH = 3
_MASK_RATIO = 0.3
_REORDER_RATIO = 0.5
_MIN_W = 2
_MAX_W = 5
_COLS = 1024
_CHUNK = 128


def _splat(v, i):
    idx = jnp.full((16, 1), i, jnp.int32)
    dn = lax.GatherDimensionNumbers(
        offset_dims=(), collapsed_slice_dims=(0,), start_index_map=(0,))
    return lax.gather(v, idx, dn, slice_sizes=(1,),
                      mode=lax.GatherScatterMode.PROMISE_IN_BOUNDS)


def _crop_sc_body(seq_hbm, start_hbm, clen_hbm, out_hbm,
                  inbuf, outbuf, startv, clenv):
    B, L = seq_hbm.shape
    nw = 32
    rows_pw = B // nw
    wid = lax.axis_index("s") * 2 + lax.axis_index("c")
    wbase = wid * rows_pw
    pltpu.sync_copy(start_hbm.at[pl.ds(wbase, rows_pw)], startv)
    pltpu.sync_copy(clen_hbm.at[pl.ds(wbase, rows_pw)], clenv)
    lane0 = lax.iota(jnp.int32, 16)
    nsteps = (L + 15) // 16

    def do_chunk(ci, _):
        cbase = wbase + ci * _CHUNK
        pltpu.sync_copy(seq_hbm.at[pl.ds(cbase, _CHUNK)], inbuf)

        def do_group(gi, __):
            st16 = startv[pl.ds(ci * _CHUNK + gi * 16, 16)]
            cl16 = clenv[pl.ds(ci * _CHUNK + gi * 16, 16)]
            for rr in range(16):
                sv = _splat(st16, rr)
                cv = _splat(cl16, rr)
                rv = jnp.full((16,), gi * 16 + rr, jnp.int32)
                for i in range(nsteps):
                    lane = lane0 + 16 * i
                    col = jnp.minimum(sv + lane, L - 1)
                    g = plsc.load_gather(inbuf, [rv, col])
                    outv = jnp.where(lane < cv, g, 0)
                    if 16 * (i + 1) <= L:
                        plsc.store_scatter(outbuf, [rv, lane], outv)
                    else:
                        plsc.store_scatter(outbuf, [rv, lane], outv,
                                           mask=lane < L)
            return __

        lax.fori_loop(0, _CHUNK // 16, do_group, 0)
        pltpu.sync_copy(outbuf, out_hbm.at[pl.ds(cbase, _CHUNK)])
        return _

    lax.fori_loop(0, rows_pw // _CHUNK, do_chunk, 0)


def _crop_sc(item_seq, start_eff, clen_eff):
    B, L = item_seq.shape
    mesh = plsc.VectorSubcoreMesh(core_axis_name="c", subcore_axis_name="s")
    f = functools.partial(
        pl.kernel,
        mesh=mesh,
        compiler_params=pltpu.CompilerParams(needs_layout_passes=False),
        out_type=jax.ShapeDtypeStruct((B, L), jnp.int32),
        scratch_types=[
            pltpu.VMEM((_CHUNK, L), jnp.int32),
            pltpu.VMEM((_CHUNK, L), jnp.int32),
            pltpu.VMEM((B // 32,), jnp.int32),
            pltpu.VMEM((B // 32,), jnp.int32),
        ],
    )(_crop_sc_body)
    return f(item_seq, start_eff, clen_eff)


def _aug_body(seq_ref, len_ref, sc_ref, uw_ref, us_ref, r8_ref,
              ua_ref, mask_ref, reord_ref):
    seq = seq_ref[...]
    lens = len_ref[...]
    L, C = seq.shape
    pos = lax.broadcasted_iota(jnp.int32, (L, C), 0)
    zero = jnp.zeros_like(seq)

    valid = (seq != 0) & (pos < lens)
    vi = valid.astype(jnp.int32)
    n_valid = jnp.sum(vi, axis=0, keepdims=True)
    num_to_mask = jnp.minimum(
        jnp.maximum(1, (n_valid.astype(jnp.float32) * _MASK_RATIO).astype(jnp.int32)),
        n_valid)
    sc = jnp.where(valid, sc_ref[...], 2.0)
    kp1 = jnp.clip(num_to_mask - 1, 0, L - 1) + 1
    lo = jnp.zeros_like(lens)
    hi = jnp.full_like(lens, 1 << 23)
    for _ in range(24):
        mid = (lo + hi) >> 1
        t = lax.bitcast_convert_type(mid + 0x3F800000, jnp.float32) - 1.0
        cnt = jnp.sum((sc <= t).astype(jnp.int32), axis=0, keepdims=True)
        ge = cnt >= kp1
        hi = jnp.where(ge, mid, hi)
        lo = jnp.where(ge, lo, mid + 1)
    thresh = lax.bitcast_convert_type(hi + 0x3F800000, jnp.float32) - 1.0
    apply_m = (lens > 1) & (n_valid > 0)
    to_mask = valid & (sc <= thresh)
    mask_ref[...] = jnp.where(apply_m & to_mask, zero, seq)

    max_possible = jnp.minimum(n_valid, _MAX_W)
    w = _MIN_W + jnp.floor(
        uw_ref[...] * jnp.maximum(max_possible - _MIN_W + 1, 1).astype(jnp.float32)
    ).astype(jnp.int32)
    w = jnp.clip(w, _MIN_W, jnp.maximum(max_possible, _MIN_W))
    max_start2 = jnp.maximum(n_valid - w + 1, 1)
    s = jnp.minimum(
        jnp.floor(us_ref[...] * max_start2.astype(jnp.float32)).astype(jnp.int32),
        max_start2 - 1)
    applied = (ua_ref[...] <= _REORDER_RATIO) & (lens > _MIN_W) & (n_valid >= _MIN_W)

    c = vi
    sh = 1
    while sh < L:
        moved = jnp.concatenate(
            [jnp.zeros((sh, C), jnp.int32), c[:L - sh, :]], axis=0)
        c = c + moved
        sh <<= 1
    excl = c - vi

    posk, valk = [], []
    for k in range(_MAX_W):
        hit = valid & (excl == s + k)
        posk.append(jnp.sum(jnp.where(hit, pos, 0), axis=0, keepdims=True))
        valk.append(jnp.sum(jnp.where(hit, seq, 0), axis=0, keepdims=True))

    r8 = r8_ref[...]
    rk = [jnp.where(k < w, r8[k:k + 1, :], 2.0) for k in range(_MAX_W)]
    ranks = []
    for i in range(_MAX_W):
        acc = jnp.zeros_like(lens)
        for j in range(_MAX_W):
            if j == i:
                continue
            cmp = (rk[j] <= rk[i]) if j < i else (rk[j] < rk[i])
            acc = acc + cmp.astype(jnp.int32)
        ranks.append(acc)

    out_r = seq
    for p in range(_MAX_W):
        vsrc = jnp.zeros_like(lens)
        for i in range(_MAX_W):
            vsrc = vsrc + jnp.where(ranks[i] == p, valk[i], 0)
        cond = applied & (p < w) & (pos == posk[p])
        out_r = jnp.where(cond, vsrc, out_r)
    reord_ref[...] = out_r


def kernel(item_seq, item_seq_len):
    B, L = item_seq.shape
    C = _COLS

    key = jax.random.key(42)
    kc, km, kr = jax.random.split(key, 3)
    u_crop = jax.random.uniform(kc, (B,))
    scores = jax.random.uniform(km, (B, L))
    k1, k2, k3, k4 = jax.random.split(kr, 4)
    u_w = jax.random.uniform(k1, (B,))
    u_s = jax.random.uniform(k2, (B,))
    r = jax.random.uniform(k3, (B, _MAX_W))
    u_apply = jax.random.uniform(k4, (B,))
    r8 = jnp.pad(r.T, ((0, 8 - _MAX_W), (0, 0)), constant_values=2.0)

    lens1 = item_seq_len.astype(jnp.int32)
    crop_len = jnp.maximum(_MIN_LENGTH, (lens1.astype(jnp.float32) * _CROP_RATIO)
                           .astype(jnp.int32))
    crop_len = jnp.minimum(crop_len, lens1)
    max_start = jnp.maximum(lens1 - crop_len + 1, 1)
    start = jnp.minimum(
        jnp.floor(u_crop * max_start.astype(jnp.float32)).astype(jnp.int32),
        max_start - 1)
    apply_c = lens1 > _MIN_LENGTH
    start_eff = jnp.where(apply_c, start, 0)
    clen_eff = jnp.where(apply_c, crop_len, L)
    cl = jnp.where(apply_c, crop_len, lens1)

    cs = _crop_sc(item_seq, start_eff, clen_eff)

    seq_t = item_seq.T
    scores_t = scores.T
    lens = lens1.reshape(1, B)
    row = lambda x: x.reshape(1, B)

    big_spec = pl.BlockSpec((L, C), lambda i: (0, i))
    one_spec = pl.BlockSpec((1, C), lambda i: (0, i))
    r8_spec = pl.BlockSpec((8, C), lambda i: (0, i))

    ms, rs = pl.pallas_call(
        _aug_body,
        grid=(B // C,),
        in_specs=[big_spec, one_spec, big_spec, one_spec,
                  one_spec, r8_spec, one_spec],
        out_specs=[big_spec, big_spec],
        out_shape=[
            jax.ShapeDtypeStruct((L, B), jnp.int32),
            jax.ShapeDtypeStruct((L, B), jnp.int32),
        ],
        compiler_params=pltpu.CompilerParams(
            dimension_semantics=("parallel",)),
    )(seq_t, lens, scores_t, row(u_w), row(u_s), r8, row(u_apply))

    cs, _ = lax.optimization_barrier((cs, ms[0, 0]))

    return cs, cl, ms.T, lens1, rs.T, lens1
